# Initial kernel scaffold; baseline (speedup 1.0000x reference)
import jax
import jax.numpy as jnp
from jax.experimental import pallas as pl


def kernel(x, edge_index, t, batch_indices, params):
    raise NotImplementedError("write your pallas kernel here")


# speedup vs baseline: 2.4342x; 2.4342x over previous
import functools

import jax
import jax.numpy as jnp
from jax import lax
from jax.experimental import pallas as pl
from jax.experimental.pallas import tpu as pltpu
from jax.experimental.pallas import tpu_sc as plsc

_N = 10000
_E = 320000
_G = 64
_H = 128
_TT = 1000.0

_NC = 2
_NS = 16
_NW = _NC * _NS
_PER_W = _E // _NW
_CH = 80
_NIT = _PER_W // _CH
_NR = _N // _NS

_EB = 2000
_NB = 2000

_F32 = jnp.float32


def _silu(v):
    return v * jax.nn.sigmoid(v)


def _b(v):
    return v.astype(jnp.bfloat16).astype(_F32)


def _sc_gather(tables, ii, jj):
    nt = len(tables)
    mesh = plsc.VectorSubcoreMesh(core_axis_name="c", subcore_axis_name="s")

    @functools.partial(
        pl.kernel,
        out_type=tuple(jax.ShapeDtypeStruct((_E, _H), _F32)
                       for _ in range(2 * nt)),
        mesh=mesh,
        scratch_types=[
            pltpu.VMEM((2, _CH), jnp.int32),
            pltpu.VMEM((2 * nt, _CH, _H), _F32),
            pltpu.SemaphoreType.DMA,
        ],
    )
    def run(*refs):
        tabs = refs[:nt]
        ii_h, jj_h = refs[nt], refs[nt + 1]
        outs = refs[nt + 2:3 * nt + 2]
        idx_v, buf_v, sem = refs[3 * nt + 2:]
        wid = lax.axis_index("s") * _NC + lax.axis_index("c")
        base = wid * _PER_W

        def body(k, _):
            off = base + k * _CH
            pltpu.sync_copy(ii_h.at[pl.ds(off, _CH)], idx_v.at[0])
            pltpu.sync_copy(jj_h.at[pl.ds(off, _CH)], idx_v.at[1])
            cps = []
            for ti in range(nt):
                cps.append(pltpu.async_copy(
                    tabs[ti].at[idx_v.at[0]], buf_v.at[2 * ti], sem))
                cps.append(pltpu.async_copy(
                    tabs[ti].at[idx_v.at[1]], buf_v.at[2 * ti + 1], sem))
            for c in cps:
                c.wait()
            for m in range(2 * nt):
                pltpu.sync_copy(buf_v.at[m], outs[m].at[pl.ds(off, _CH)])
            return 0

        lax.fori_loop(0, _NIT, body, 0)

    return run(*tables, ii, jj)


def _sc_scatter(msg, dvec, ii, jj):
    mesh = plsc.VectorSubcoreMesh(core_axis_name="c", subcore_axis_name="s")

    @functools.partial(
        pl.kernel,
        out_type=(jax.ShapeDtypeStruct((_NC, _N, _H), _F32),
                  jax.ShapeDtypeStruct((_NC, _N, _H), _F32)),
        mesh=mesh,
        scratch_types=[
            pltpu.VMEM((2, _CH), jnp.int32),
            pltpu.VMEM((_CH, _H), _F32),
            pltpu.VMEM((16, _H), _F32),
            pltpu.VMEM_SHARED((_N, _H), _F32),
            pltpu.SemaphoreType.DMA,
        ],
    )
    def run(msg_h, dvec_h, ii_h, jj_h, accf_h, accx_h,
            idx_v, row_v, zb_v, acc_s, sem):
        cid = lax.axis_index("c")
        sid = lax.axis_index("s")
        wid = sid * _NC + cid

        def zrow(r, _):
            for cb in range(8):
                zb_v[r, pl.ds(cb * 16, 16)] = jnp.zeros((16,), _F32)
            return 0

        lax.fori_loop(0, 16, zrow, 0)
        ntrips = jnp.where(sid == _NS - 1, 40, 39)
        nrows = jnp.where(sid == _NS - 1, 640, 624)
        base_r = sid * 624
        base = wid * _PER_W

        def zero_acc():
            def zcopy(tk, _):
                pltpu.sync_copy(zb_v, acc_s.at[pl.ds(base_r + tk * 16, 16)])
                return 0
            lax.fori_loop(0, ntrips, zcopy, 0)

        zero_acc()
        plsc.subcore_barrier()

        def body1(k, _):
            off = base + k * _CH
            pltpu.sync_copy(ii_h.at[pl.ds(off, _CH)], idx_v.at[0])
            pltpu.sync_copy(msg_h.at[pl.ds(off, _CH)], row_v)
            pltpu.sync_copy(row_v, acc_s.at[idx_v.at[0]], add=True)
            return 0

        lax.fori_loop(0, _NIT, body1, 0)
        plsc.subcore_barrier()

        @pl.when(sid < _NS - 1)
        def _dump_f():
            pltpu.sync_copy(acc_s.at[pl.ds(base_r, 624)],
                            accf_h.at[cid, pl.ds(base_r, 624)])

        @pl.when(sid == _NS - 1)
        def _dump_f_last():
            pltpu.sync_copy(acc_s.at[pl.ds(base_r, 640)],
                            accf_h.at[cid, pl.ds(base_r, 640)])

        plsc.subcore_barrier()

        zero_acc()
        plsc.subcore_barrier()

        def body2(k, _):
            off = base + k * _CH
            pltpu.sync_copy(jj_h.at[pl.ds(off, _CH)], idx_v.at[1])
            pltpu.sync_copy(dvec_h.at[pl.ds(off, _CH)], row_v)
            pltpu.sync_copy(row_v, acc_s.at[idx_v.at[1]], add=True)
            return 0

        lax.fori_loop(0, _NIT, body2, 0)
        plsc.subcore_barrier()

        @pl.when(sid < _NS - 1)
        def _dump_x():
            pltpu.sync_copy(acc_s.at[pl.ds(base_r, 624)],
                            accx_h.at[cid, pl.ds(base_r, 624)])

        @pl.when(sid == _NS - 1)
        def _dump_x_last():
            pltpu.sync_copy(acc_s.at[pl.ds(base_r, 640)],
                            accx_h.at[cid, pl.ds(base_r, 640)])

    return run(msg, dvec, ii, jj)


def _tc_edge0(gxi, gxj, w2, wd2, vecs):
    def kern(gi_r, gj_r, w2_r, wd2_r, v_r, msg_r, dvec_r, dist_r):
        rel8 = gj_r[:, 0:8] - gi_r[:, 0:8]
        dist = jnp.sqrt(jnp.sum(rel8 * rel8, axis=1, keepdims=True))
        fi = _b(gi_r[:, 8:9])
        fj = _b(gj_r[:, 8:9])
        db = _b(dist)
        w1a, w1b, w1c, w1d, b1, b2 = (v_r[0:1], v_r[1:2], v_r[2:3], v_r[3:4],
                                      v_r[4:5], v_r[5:6])
        wd1a, wd1b, wd1c, wd1d, bd1, bd2 = (v_r[6:7], v_r[7:8], v_r[8:9],
                                            v_r[9:10], v_r[10:11], v_r[11:12])
        wi, wd3 = v_r[12:13], v_r[13:14]
        bi = v_r[14:15, 0:1]
        bd3 = v_r[14:15, 1:2]
        mm = lambda a, b: jnp.dot(a, b, preferred_element_type=_F32)
        m1 = _silu(fi * _b(w1a) + fj * _b(w1b) + db * _b(w1c)
                   + db * _b(w1d) + b1)
        m2 = _silu(mm(m1, w2_r[...]) + b2)
        e = jax.nn.sigmoid(jnp.sum(_b(m2) * _b(wi), axis=1, keepdims=True)
                           + bi)
        msg_r[...] = e * m2
        d1 = _silu(fj * _b(wd1a) + fi * _b(wd1b) + db * _b(wd1c)
                   + db * _b(wd1d) + bd1)
        d2 = _silu(mm(d1, wd2_r[...]) + bd2)
        dw = jnp.sum(_b(d2) * _b(wd3), axis=1, keepdims=True) + bd3
        lane = lax.broadcasted_iota(jnp.int32, (_EB, _H), 1)
        rel128 = jnp.where(lane < 3, gj_r[...] - gi_r[...], 0.0)
        dvec_r[...] = rel128 / (dist + 1.0) * dw
        dist_r[...] = dist

    grid = (_E // _EB,)
    return pl.pallas_call(
        kern,
        grid=grid,
        in_specs=[
            pl.BlockSpec((_EB, _H), lambda b: (b, 0)),
            pl.BlockSpec((_EB, _H), lambda b: (b, 0)),
            pl.BlockSpec((_H, _H), lambda b: (0, 0)),
            pl.BlockSpec((_H, _H), lambda b: (0, 0)),
            pl.BlockSpec((16, _H), lambda b: (0, 0)),
        ],
        out_specs=[
            pl.BlockSpec((_EB, _H), lambda b: (b, 0)),
            pl.BlockSpec((_EB, _H), lambda b: (b, 0)),
            pl.BlockSpec((_EB, 1), lambda b: (b, 0)),
        ],
        out_shape=[
            jax.ShapeDtypeStruct((_E, _H), _F32),
            jax.ShapeDtypeStruct((_E, _H), _F32),
            jax.ShapeDtypeStruct((_E, 1), _F32),
        ],
    )(gxi, gxj, w2, wd2, vecs)


def _tc_edge(gfi, gfj, gxi, gxj, d0, w1a, w1b, w2, wd1a, wd1b, wd2, vecs):
    def kern(gfi_r, gfj_r, gxi_r, gxj_r, d0_r, w1a_r, w1b_r, w2_r, wd1a_r,
             wd1b_r, wd2_r, v_r, msg_r, dvec_r):
        rel = gxj_r[:, 0:16] - gxi_r[:, 0:16]
        dist = jnp.sqrt(jnp.sum(rel * rel, axis=1, keepdims=True))
        fi = gfi_r[...]
        fj = gfj_r[...]
        d0 = _b(d0_r[...])
        db = _b(dist)
        w1c, w1d, b1, b2 = v_r[0:1], v_r[1:2], v_r[2:3], v_r[3:4]
        wd1c, wd1d, bd1, bd2 = v_r[4:5], v_r[5:6], v_r[6:7], v_r[7:8]
        wi, wd3 = v_r[8:9], v_r[9:10]
        bi = v_r[10:11, 0:1]
        bd3 = v_r[10:11, 1:2]
        mm = lambda a, b: jnp.dot(a, b, preferred_element_type=_F32)
        m1 = _silu(mm(fi, w1a_r[...]) + mm(fj, w1b_r[...])
                   + db * _b(w1c) + d0 * _b(w1d) + b1)
        m2 = _silu(mm(m1, w2_r[...]) + b2)
        e = jax.nn.sigmoid(jnp.sum(_b(m2) * _b(wi), axis=1, keepdims=True)
                           + bi)
        msg_r[...] = e * m2
        d1 = _silu(mm(fj, wd1a_r[...]) + mm(fi, wd1b_r[...])
                   + db * _b(wd1c) + d0 * _b(wd1d) + bd1)
        d2 = _silu(mm(d1, wd2_r[...]) + bd2)
        dw = jnp.sum(_b(d2) * _b(wd3), axis=1, keepdims=True) + bd3
        lane = lax.broadcasted_iota(jnp.int32, (_EB, _H), 1)
        rel128 = jnp.where(lane < 3, gxj_r[...] - gxi_r[...], 0.0)
        dvec_r[...] = rel128 / (dist + 1.0) * dw

    grid = (_E // _EB,)
    espec = pl.BlockSpec((_EB, _H), lambda b: (b, 0))
    wspec = pl.BlockSpec((_H, _H), lambda b: (0, 0))
    return pl.pallas_call(
        kern,
        grid=grid,
        in_specs=[
            espec, espec, espec, espec,
            pl.BlockSpec((_EB, 1), lambda b: (b, 0)),
            wspec, wspec, wspec, wspec, wspec, wspec,
            pl.BlockSpec((16, _H), lambda b: (0, 0)),
        ],
        out_specs=[espec, espec],
        out_shape=[
            jax.ShapeDtypeStruct((_E, _H), _F32),
            jax.ShapeDtypeStruct((_E, _H), _F32),
        ],
    )(gfi, gfj, gxi, gxj, d0, w1a, w1b, w2, wd1a, wd1b, wd2, vecs)


def _node_call(kern, tabx, accf, accx, weights, nvecs, extra_specs):
    grid = (_N // _NB,)
    wspec = pl.BlockSpec((_H, _H), lambda b: (0, 0))
    nspec = pl.BlockSpec((_NB, _H), lambda b: (b, 0))
    return pl.pallas_call(
        kern,
        grid=grid,
        in_specs=extra_specs + [
            nspec,
            pl.BlockSpec((_NC, _NB, _H), lambda b: (0, b, 0)),
            pl.BlockSpec((_NC, _NB, _H), lambda b: (0, b, 0)),
        ] + [wspec] * len(weights) + [
            pl.BlockSpec((8, _H), lambda b: (0, 0)),
        ],
        out_specs=[nspec, nspec],
        out_shape=[
            jax.ShapeDtypeStruct((_N, _H), _F32),
            jax.ShapeDtypeStruct((_N, _H), _F32),
        ],
    )


def _tc_node0(tabx, accf, accx, wf1b, wf2, nvecs):
    def kern(tab_r, accf_r, accx_r, wf1b_r, wf2_r, nv_r, outf_r, outx_r):
        sum_m = accf_r[0] + accf_r[1]
        feats = tab_r[:, 8:9]
        wf1a, bf1, bf2 = nv_r[0:1], nv_r[1:2], nv_r[2:3]
        mm = lambda a, b: jnp.dot(a, b, preferred_element_type=_F32)
        h1 = _silu(_b(feats) * _b(wf1a) + mm(sum_m, wf1b_r[...]) + bf1)
        h2 = _silu(mm(h1, wf2_r[...]) + bf2)
        lane = lax.broadcasted_iota(jnp.int32, (_NB, _H), 1)
        xold = jnp.where(lane < 3, tab_r[...], 0.0)
        outf_r[...] = h2
        outx_r[...] = xold + accx_r[0] + accx_r[1]

    return _node_call(kern, tabx, accf, accx, (wf1b, wf2), nvecs,
                      [])(tabx, accf, accx, wf1b, wf2, nvecs)


def _tc_node(tabf, tabx, accf, accx, wf1a, wf1b, wf2, nvecs):
    def kern(tabf_r, tabx_r, accf_r, accx_r, wf1a_r, wf1b_r, wf2_r, nv_r,
             outf_r, outx_r):
        sum_m = accf_r[0] + accf_r[1]
        bf1, bf2 = nv_r[0:1], nv_r[1:2]
        mm = lambda a, b: jnp.dot(a, b, preferred_element_type=_F32)
        h1 = _silu(mm(tabf_r[...], wf1a_r[...]) + mm(sum_m, wf1b_r[...])
                   + bf1)
        h2 = _silu(mm(h1, wf2_r[...]) + bf2)
        outf_r[...] = h2
        outx_r[...] = tabx_r[...] + accx_r[0] + accx_r[1]

    nspec = pl.BlockSpec((_NB, _H), lambda b: (b, 0))
    return _node_call(kern, tabx, accf, accx, (wf1a, wf1b, wf2), nvecs,
                      [nspec])(tabf, tabx, accf, accx, wf1a, wf1b, wf2,
                               nvecs)


def _tc_post(tabx, x0p, batch):
    def kern(tab_r, x0_r, b_r, out_r):
        vel = tab_r[:, 0:16] - x0_r[...]
        gid = lax.broadcasted_iota(jnp.int32, (_N, _G), 1)
        oh = (b_r[...] == gid).astype(_F32)
        sums = lax.dot_general(oh, vel, (((0,), (0,)), ((), ())),
                               precision=lax.Precision.HIGHEST,
                               preferred_element_type=_F32)
        counts = jnp.sum(oh, axis=0)
        maxc = jnp.max(counts)
        com = sums / maxc
        out_r[...] = vel - jnp.dot(oh, com, precision=lax.Precision.HIGHEST,
                                   preferred_element_type=_F32)

    return pl.pallas_call(
        kern,
        in_specs=[
            pl.BlockSpec((_N, _H), lambda: (0, 0)),
            pl.BlockSpec((_N, 16), lambda: (0, 0)),
            pl.BlockSpec((_N, 1), lambda: (0, 0)),
        ],
        out_specs=pl.BlockSpec((_N, 16), lambda: (0, 0)),
        out_shape=jax.ShapeDtypeStruct((_N, 16), _F32),
    )(tabx, x0p, batch)


def _pack_layer0(p):
    (W1, b1), (W2, b2) = p["msg"]
    (Wi, bi), = p["inf"]
    (Wd1, bd1), (Wd2, bd2), (Wd3, bd3) = p["dirw"]
    scal = jnp.zeros((_H,), _F32).at[0].set(bi[0]).at[1].set(bd3[0])
    vecs = jnp.stack([
        W1[0], W1[1], W1[2], W1[3], b1, b2,
        Wd1[0], Wd1[1], Wd1[2], Wd1[3], bd1, bd2,
        Wi[:, 0], Wd3[:, 0], scal,
        jnp.zeros((_H,), _F32),
    ])
    (Wf1, bf1), (Wf2, bf2) = p["feat"]
    nvecs = jnp.stack([Wf1[0], bf1, bf2] + [jnp.zeros((_H,), _F32)] * 5)
    return W2, Wd2, vecs, Wf1[1:129], Wf2, nvecs


def _pack_layer(p):
    (W1, b1), (W2, b2) = p["msg"]
    (Wi, bi), = p["inf"]
    (Wd1, bd1), (Wd2, bd2), (Wd3, bd3) = p["dirw"]
    scal = jnp.zeros((_H,), _F32).at[0].set(bi[0]).at[1].set(bd3[0])
    vecs = jnp.stack([
        W1[256], W1[257], b1, b2,
        Wd1[256], Wd1[257], bd1, bd2,
        Wi[:, 0], Wd3[:, 0], scal,
    ] + [jnp.zeros((_H,), _F32)] * 5)
    (Wf1, bf1), (Wf2, bf2) = p["feat"]
    nvecs = jnp.stack([bf1, bf2] + [jnp.zeros((_H,), _F32)] * 6)
    return (W1[0:128], W1[128:256], W2, Wd1[0:128], Wd1[128:256], Wd2, vecs,
            Wf1[0:128], Wf1[128:256], Wf2, nvecs)


def kernel(x, edge_index, t, batch_indices, params):
    x = x.astype(_F32)
    ii = edge_index[1]
    jj = edge_index[0]
    zeros = lambda w: jnp.zeros((_N, w), _F32)
    x0p = jnp.concatenate([x, zeros(13)], axis=1)
    tabx = jnp.concatenate([x, zeros(5), (t / _TT)[:, None], zeros(119)],
                           axis=1)

    W2, Wd2, vecs0, Wf1b, Wf2, nvecs0 = _pack_layer0(params[0])
    gxi, gxj = _sc_gather([tabx], ii, jj)
    msg, dvec, dist0 = _tc_edge0(gxi, gxj, W2, Wd2, vecs0)
    accf, accx = _sc_scatter(msg, dvec, ii, jj)
    tabf, tabx = _tc_node0(tabx, accf, accx, Wf1b, Wf2, nvecs0)

    for p in params[1:]:
        (w1a, w1b, w2, wd1a, wd1b, wd2, vecs,
         wf1a, wf1b, wf2, nvecs) = _pack_layer(p)
        gfi, gfj, gxi, gxj = _sc_gather([tabf, tabx], ii, jj)
        msg, dvec = _tc_edge(gfi, gfj, gxi, gxj, dist0, w1a, w1b, w2, wd1a,
                             wd1b, wd2, vecs)
        accf, accx = _sc_scatter(msg, dvec, ii, jj)
        tabf, tabx = _tc_node(tabf, tabx, accf, accx, wf1a, wf1b, wf2, nvecs)

    vel = _tc_post(tabx, x0p, batch_indices[:, None].astype(jnp.int32))
    return vel[:, 0:3]


# speedup vs baseline: 2.9272x; 1.2025x over previous
import functools

import jax
import jax.numpy as jnp
from jax import lax
from jax.experimental import pallas as pl
from jax.experimental.pallas import tpu as pltpu
from jax.experimental.pallas import tpu_sc as plsc

_N = 10000
_E = 320000
_G = 64
_H = 128
_TT = 1000.0

_NC = 2
_NS = 16
_NW = _NC * _NS
_PER_W = _E // _NW
_CH = 80
_NIT = _PER_W // _CH
_NR = _N // _NS

_EB = 4000
_NB = 2000

_F32 = jnp.float32


def _silu(v):
    return v * jax.nn.sigmoid(v)


def _b(v):
    return v.astype(jnp.bfloat16).astype(_F32)


def _sc_gather(tables, ii, jj):
    nt = len(tables)
    mesh = plsc.VectorSubcoreMesh(core_axis_name="c", subcore_axis_name="s")

    @functools.partial(
        pl.kernel,
        out_type=tuple(jax.ShapeDtypeStruct((_E, _H), _F32)
                       for _ in range(2 * nt)),
        mesh=mesh,
        scratch_types=[
            pltpu.VMEM((2, _CH), jnp.int32),
            pltpu.VMEM((2 * nt, _CH, _H), _F32),
            pltpu.SemaphoreType.DMA,
        ],
    )
    def run(*refs):
        tabs = refs[:nt]
        ii_h, jj_h = refs[nt], refs[nt + 1]
        outs = refs[nt + 2:3 * nt + 2]
        idx_v, buf_v, sem = refs[3 * nt + 2:]
        wid = lax.axis_index("s") * _NC + lax.axis_index("c")
        base = wid * _PER_W

        def body(k, _):
            off = base + k * _CH
            pltpu.sync_copy(ii_h.at[pl.ds(off, _CH)], idx_v.at[0])
            pltpu.sync_copy(jj_h.at[pl.ds(off, _CH)], idx_v.at[1])
            cps = []
            for ti in range(nt):
                cps.append(pltpu.async_copy(
                    tabs[ti].at[idx_v.at[0]], buf_v.at[2 * ti], sem))
                cps.append(pltpu.async_copy(
                    tabs[ti].at[idx_v.at[1]], buf_v.at[2 * ti + 1], sem))
            for c in cps:
                c.wait()
            for m in range(2 * nt):
                pltpu.sync_copy(buf_v.at[m], outs[m].at[pl.ds(off, _CH)])
            return 0

        lax.fori_loop(0, _NIT, body, 0)

    return run(*tables, ii, jj)


def _sc_scatter(msg, dvec, ii, jj):
    mesh = plsc.VectorSubcoreMesh(core_axis_name="c", subcore_axis_name="s")

    @functools.partial(
        pl.kernel,
        out_type=(jax.ShapeDtypeStruct((_NC, _N, _H), _F32),
                  jax.ShapeDtypeStruct((_NC, _N, _H), _F32)),
        mesh=mesh,
        scratch_types=[
            pltpu.VMEM((2, _CH), jnp.int32),
            pltpu.VMEM((_CH, _H), _F32),
            pltpu.VMEM((16, _H), _F32),
            pltpu.VMEM_SHARED((_N, _H), _F32),
            pltpu.SemaphoreType.DMA,
        ],
    )
    def run(msg_h, dvec_h, ii_h, jj_h, accf_h, accx_h,
            idx_v, row_v, zb_v, acc_s, sem):
        cid = lax.axis_index("c")
        sid = lax.axis_index("s")
        wid = sid * _NC + cid

        def zrow(r, _):
            for cb in range(8):
                zb_v[r, pl.ds(cb * 16, 16)] = jnp.zeros((16,), _F32)
            return 0

        lax.fori_loop(0, 16, zrow, 0)
        ntrips = jnp.where(sid == _NS - 1, 40, 39)
        nrows = jnp.where(sid == _NS - 1, 640, 624)
        base_r = sid * 624
        base = wid * _PER_W

        def zero_acc():
            def zcopy(tk, _):
                pltpu.sync_copy(zb_v, acc_s.at[pl.ds(base_r + tk * 16, 16)])
                return 0
            lax.fori_loop(0, ntrips, zcopy, 0)

        zero_acc()
        plsc.subcore_barrier()

        def body1(k, _):
            off = base + k * _CH
            pltpu.sync_copy(ii_h.at[pl.ds(off, _CH)], idx_v.at[0])
            pltpu.sync_copy(msg_h.at[pl.ds(off, _CH)], row_v)
            pltpu.sync_copy(row_v, acc_s.at[idx_v.at[0]], add=True)
            return 0

        lax.fori_loop(0, _NIT, body1, 0)
        plsc.subcore_barrier()

        @pl.when(sid < _NS - 1)
        def _dump_f():
            pltpu.sync_copy(acc_s.at[pl.ds(base_r, 624)],
                            accf_h.at[cid, pl.ds(base_r, 624)])

        @pl.when(sid == _NS - 1)
        def _dump_f_last():
            pltpu.sync_copy(acc_s.at[pl.ds(base_r, 640)],
                            accf_h.at[cid, pl.ds(base_r, 640)])

        plsc.subcore_barrier()

        zero_acc()
        plsc.subcore_barrier()

        def body2(k, _):
            off = base + k * _CH
            pltpu.sync_copy(jj_h.at[pl.ds(off, _CH)], idx_v.at[1])
            pltpu.sync_copy(dvec_h.at[pl.ds(off, _CH)], row_v)
            pltpu.sync_copy(row_v, acc_s.at[idx_v.at[1]], add=True)
            return 0

        lax.fori_loop(0, _NIT, body2, 0)
        plsc.subcore_barrier()

        @pl.when(sid < _NS - 1)
        def _dump_x():
            pltpu.sync_copy(acc_s.at[pl.ds(base_r, 624)],
                            accx_h.at[cid, pl.ds(base_r, 624)])

        @pl.when(sid == _NS - 1)
        def _dump_x_last():
            pltpu.sync_copy(acc_s.at[pl.ds(base_r, 640)],
                            accx_h.at[cid, pl.ds(base_r, 640)])

    return run(msg, dvec, ii, jj)


def _tc_edge0(gxi, gxj, w2, wd2, vecs):
    def kern(gi_r, gj_r, w2_r, wd2_r, v_r, msg_r, dvec_r, dist_r):
        rel8 = gj_r[:, 0:8] - gi_r[:, 0:8]
        dist = jnp.sqrt(jnp.sum(rel8 * rel8, axis=1, keepdims=True))
        fi = _b(gi_r[:, 8:9])
        fj = _b(gj_r[:, 8:9])
        db = _b(dist)
        w1a, w1b, w1c, w1d, b1, b2 = (v_r[0:1], v_r[1:2], v_r[2:3], v_r[3:4],
                                      v_r[4:5], v_r[5:6])
        wd1a, wd1b, wd1c, wd1d, bd1, bd2 = (v_r[6:7], v_r[7:8], v_r[8:9],
                                            v_r[9:10], v_r[10:11], v_r[11:12])
        wi, wd3 = v_r[12:13], v_r[13:14]
        bi = v_r[14:15, 0:1]
        bd3 = v_r[14:15, 1:2]
        mm = lambda a, b: jnp.dot(a, b, preferred_element_type=_F32)
        m1 = _silu(fi * _b(w1a) + fj * _b(w1b) + db * _b(w1c)
                   + db * _b(w1d) + b1)
        m2 = _silu(mm(m1, w2_r[...]) + b2)
        e = jax.nn.sigmoid(jnp.sum(_b(m2) * _b(wi), axis=1, keepdims=True)
                           + bi)
        msg_r[...] = e * m2
        d1 = _silu(fj * _b(wd1a) + fi * _b(wd1b) + db * _b(wd1c)
                   + db * _b(wd1d) + bd1)
        d2 = _silu(mm(d1, wd2_r[...]) + bd2)
        dw = jnp.sum(_b(d2) * _b(wd3), axis=1, keepdims=True) + bd3
        lane = lax.broadcasted_iota(jnp.int32, (_EB, _H), 1)
        rel128 = jnp.where(lane < 3, gj_r[...] - gi_r[...], 0.0)
        dvec_r[...] = rel128 / (dist + 1.0) * dw
        dist_r[...] = dist

    grid = (_E // _EB,)
    return pl.pallas_call(
        kern,
        grid=grid,
        in_specs=[
            pl.BlockSpec((_EB, _H), lambda b: (b, 0)),
            pl.BlockSpec((_EB, _H), lambda b: (b, 0)),
            pl.BlockSpec((_H, _H), lambda b: (0, 0)),
            pl.BlockSpec((_H, _H), lambda b: (0, 0)),
            pl.BlockSpec((16, _H), lambda b: (0, 0)),
        ],
        out_specs=[
            pl.BlockSpec((_EB, _H), lambda b: (b, 0)),
            pl.BlockSpec((_EB, _H), lambda b: (b, 0)),
            pl.BlockSpec((_EB, 1), lambda b: (b, 0)),
        ],
        out_shape=[
            jax.ShapeDtypeStruct((_E, _H), _F32),
            jax.ShapeDtypeStruct((_E, _H), _F32),
            jax.ShapeDtypeStruct((_E, 1), _F32),
        ],
    )(gxi, gxj, w2, wd2, vecs)


def _tc_edge(gfi, gfj, gxi, gxj, d0, w1a, w1b, w2, wd1a, wd1b, wd2, vecs):
    def kern(gfi_r, gfj_r, gxi_r, gxj_r, d0_r, w1a_r, w1b_r, w2_r, wd1a_r,
             wd1b_r, wd2_r, v_r, msg_r, dvec_r):
        rel = gxj_r[:, 0:16] - gxi_r[:, 0:16]
        dist = jnp.sqrt(jnp.sum(rel * rel, axis=1, keepdims=True))
        fi = gfi_r[...]
        fj = gfj_r[...]
        d0 = _b(d0_r[...])
        db = _b(dist)
        w1c, w1d, b1, b2 = v_r[0:1], v_r[1:2], v_r[2:3], v_r[3:4]
        wd1c, wd1d, bd1, bd2 = v_r[4:5], v_r[5:6], v_r[6:7], v_r[7:8]
        wi, wd3 = v_r[8:9], v_r[9:10]
        bi = v_r[10:11, 0:1]
        bd3 = v_r[10:11, 1:2]
        mm = lambda a, b: jnp.dot(a, b, preferred_element_type=_F32)
        m1 = _silu(mm(fi, w1a_r[...]) + mm(fj, w1b_r[...])
                   + db * _b(w1c) + d0 * _b(w1d) + b1)
        m2 = _silu(mm(m1, w2_r[...]) + b2)
        e = jax.nn.sigmoid(jnp.sum(_b(m2) * _b(wi), axis=1, keepdims=True)
                           + bi)
        msg_r[...] = e * m2
        d1 = _silu(mm(fj, wd1a_r[...]) + mm(fi, wd1b_r[...])
                   + db * _b(wd1c) + d0 * _b(wd1d) + bd1)
        d2 = _silu(mm(d1, wd2_r[...]) + bd2)
        dw = jnp.sum(_b(d2) * _b(wd3), axis=1, keepdims=True) + bd3
        lane = lax.broadcasted_iota(jnp.int32, (_EB, _H), 1)
        rel128 = jnp.where(lane < 3, gxj_r[...] - gxi_r[...], 0.0)
        dvec_r[...] = rel128 / (dist + 1.0) * dw

    grid = (_E // _EB,)
    espec = pl.BlockSpec((_EB, _H), lambda b: (b, 0))
    wspec = pl.BlockSpec((_H, _H), lambda b: (0, 0))
    return pl.pallas_call(
        kern,
        grid=grid,
        in_specs=[
            espec, espec, espec, espec,
            pl.BlockSpec((_EB, 1), lambda b: (b, 0)),
            wspec, wspec, wspec, wspec, wspec, wspec,
            pl.BlockSpec((16, _H), lambda b: (0, 0)),
        ],
        out_specs=[espec, espec],
        out_shape=[
            jax.ShapeDtypeStruct((_E, _H), _F32),
            jax.ShapeDtypeStruct((_E, _H), _F32),
        ],
    )(gfi, gfj, gxi, gxj, d0, w1a, w1b, w2, wd1a, wd1b, wd2, vecs)


def _node_call(kern, tabx, accf, accx, weights, nvecs, extra_specs):
    grid = (_N // _NB,)
    wspec = pl.BlockSpec((_H, _H), lambda b: (0, 0))
    nspec = pl.BlockSpec((_NB, _H), lambda b: (b, 0))
    return pl.pallas_call(
        kern,
        grid=grid,
        in_specs=extra_specs + [
            nspec,
            pl.BlockSpec((_NC, _NB, _H), lambda b: (0, b, 0)),
            pl.BlockSpec((_NC, _NB, _H), lambda b: (0, b, 0)),
        ] + [wspec] * len(weights) + [
            pl.BlockSpec((8, _H), lambda b: (0, 0)),
        ],
        out_specs=[nspec, nspec],
        out_shape=[
            jax.ShapeDtypeStruct((_N, _H), _F32),
            jax.ShapeDtypeStruct((_N, _H), _F32),
        ],
    )


def _tc_node0(tabx, accf, accx, wf1b, wf2, nvecs):
    def kern(tab_r, accf_r, accx_r, wf1b_r, wf2_r, nv_r, outf_r, outx_r):
        sum_m = accf_r[0] + accf_r[1]
        feats = tab_r[:, 8:9]
        wf1a, bf1, bf2 = nv_r[0:1], nv_r[1:2], nv_r[2:3]
        mm = lambda a, b: jnp.dot(a, b, preferred_element_type=_F32)
        h1 = _silu(_b(feats) * _b(wf1a) + mm(sum_m, wf1b_r[...]) + bf1)
        h2 = _silu(mm(h1, wf2_r[...]) + bf2)
        lane = lax.broadcasted_iota(jnp.int32, (_NB, _H), 1)
        xold = jnp.where(lane < 3, tab_r[...], 0.0)
        outf_r[...] = h2
        outx_r[...] = xold + accx_r[0] + accx_r[1]

    return _node_call(kern, tabx, accf, accx, (wf1b, wf2), nvecs,
                      [])(tabx, accf, accx, wf1b, wf2, nvecs)


def _tc_node(tabf, tabx, accf, accx, wf1a, wf1b, wf2, nvecs):
    def kern(tabf_r, tabx_r, accf_r, accx_r, wf1a_r, wf1b_r, wf2_r, nv_r,
             outf_r, outx_r):
        sum_m = accf_r[0] + accf_r[1]
        bf1, bf2 = nv_r[0:1], nv_r[1:2]
        mm = lambda a, b: jnp.dot(a, b, preferred_element_type=_F32)
        h1 = _silu(mm(tabf_r[...], wf1a_r[...]) + mm(sum_m, wf1b_r[...])
                   + bf1)
        h2 = _silu(mm(h1, wf2_r[...]) + bf2)
        outf_r[...] = h2
        outx_r[...] = tabx_r[...] + accx_r[0] + accx_r[1]

    nspec = pl.BlockSpec((_NB, _H), lambda b: (b, 0))
    return _node_call(kern, tabx, accf, accx, (wf1a, wf1b, wf2), nvecs,
                      [nspec])(tabf, tabx, accf, accx, wf1a, wf1b, wf2,
                               nvecs)


def _tc_post(tabx, x0p, batch):
    def kern(tab_r, x0_r, b_r, out_r):
        vel = tab_r[:, 0:16] - x0_r[...]
        gid = lax.broadcasted_iota(jnp.int32, (_N, _G), 1)
        oh = (b_r[...] == gid).astype(_F32)
        sums = lax.dot_general(oh, vel, (((0,), (0,)), ((), ())),
                               precision=lax.Precision.HIGHEST,
                               preferred_element_type=_F32)
        counts = jnp.sum(oh, axis=0)
        maxc = jnp.max(counts)
        com = sums / maxc
        out_r[...] = vel - jnp.dot(oh, com, precision=lax.Precision.HIGHEST,
                                   preferred_element_type=_F32)

    return pl.pallas_call(
        kern,
        in_specs=[
            pl.BlockSpec((_N, _H), lambda: (0, 0)),
            pl.BlockSpec((_N, 16), lambda: (0, 0)),
            pl.BlockSpec((_N, 1), lambda: (0, 0)),
        ],
        out_specs=pl.BlockSpec((_N, 16), lambda: (0, 0)),
        out_shape=jax.ShapeDtypeStruct((_N, 16), _F32),
    )(tabx, x0p, batch)


def _pack_layer0(p):
    (W1, b1), (W2, b2) = p["msg"]
    (Wi, bi), = p["inf"]
    (Wd1, bd1), (Wd2, bd2), (Wd3, bd3) = p["dirw"]
    scal = jnp.zeros((_H,), _F32).at[0].set(bi[0]).at[1].set(bd3[0])
    vecs = jnp.stack([
        W1[0], W1[1], W1[2], W1[3], b1, b2,
        Wd1[0], Wd1[1], Wd1[2], Wd1[3], bd1, bd2,
        Wi[:, 0], Wd3[:, 0], scal,
        jnp.zeros((_H,), _F32),
    ])
    (Wf1, bf1), (Wf2, bf2) = p["feat"]
    nvecs = jnp.stack([Wf1[0], bf1, bf2] + [jnp.zeros((_H,), _F32)] * 5)
    return W2, Wd2, vecs, Wf1[1:129], Wf2, nvecs


def _pack_layer(p):
    (W1, b1), (W2, b2) = p["msg"]
    (Wi, bi), = p["inf"]
    (Wd1, bd1), (Wd2, bd2), (Wd3, bd3) = p["dirw"]
    scal = jnp.zeros((_H,), _F32).at[0].set(bi[0]).at[1].set(bd3[0])
    vecs = jnp.stack([
        W1[256], W1[257], b1, b2,
        Wd1[256], Wd1[257], bd1, bd2,
        Wi[:, 0], Wd3[:, 0], scal,
    ] + [jnp.zeros((_H,), _F32)] * 5)
    (Wf1, bf1), (Wf2, bf2) = p["feat"]
    nvecs = jnp.stack([bf1, bf2] + [jnp.zeros((_H,), _F32)] * 6)
    return (W1[0:128], W1[128:256], W2, Wd1[0:128], Wd1[128:256], Wd2, vecs,
            Wf1[0:128], Wf1[128:256], Wf2, nvecs)


def kernel(x, edge_index, t, batch_indices, params):
    x = x.astype(_F32)
    ii = edge_index[1]
    jj = edge_index[0]
    zeros = lambda w: jnp.zeros((_N, w), _F32)
    x0p = jnp.concatenate([x, zeros(13)], axis=1)
    tabx = jnp.concatenate([x, zeros(5), (t / _TT)[:, None], zeros(119)],
                           axis=1)

    W2, Wd2, vecs0, Wf1b, Wf2, nvecs0 = _pack_layer0(params[0])
    gxi, gxj = _sc_gather([tabx], ii, jj)
    msg, dvec, dist0 = _tc_edge0(gxi, gxj, W2, Wd2, vecs0)
    accf, accx = _sc_scatter(msg, dvec, ii, jj)
    tabf, tabx = _tc_node0(tabx, accf, accx, Wf1b, Wf2, nvecs0)

    for p in params[1:]:
        (w1a, w1b, w2, wd1a, wd1b, wd2, vecs,
         wf1a, wf1b, wf2, nvecs) = _pack_layer(p)
        gfi, gfj, gxi, gxj = _sc_gather([tabf, tabx], ii, jj)
        msg, dvec = _tc_edge(gfi, gfj, gxi, gxj, dist0, w1a, w1b, w2, wd1a,
                             wd1b, wd2, vecs)
        accf, accx = _sc_scatter(msg, dvec, ii, jj)
        tabf, tabx = _tc_node(tabf, tabx, accf, accx, wf1a, wf1b, wf2, nvecs)

    vel = _tc_post(tabx, x0p, batch_indices[:, None].astype(jnp.int32))
    return vel[:, 0:3]
